# Initial kernel scaffold; baseline (speedup 1.0000x reference)
import jax
import jax.numpy as jnp
from jax.experimental import pallas as pl


def kernel(x1, lu_idx, lu_vals, ld_idx, ld_vals, batch1, Wp1, Wg1, asrc1, adst1, Wp2, Wg2, asrc2, adst2, Wp3, Wg3, asrc3, adst3, Wp4, Wg4, asrc4, adst4):
    raise NotImplementedError("write your pallas kernel here")


# speedup vs baseline: 9.7426x; 9.7426x over previous
import functools

import jax
import jax.numpy as jnp
from jax import lax
from jax.experimental import pallas as pl
from jax.experimental.pallas import tpu as pltpu
from jax.experimental.pallas import tpu_sc as plsc

_N = 10000
_NB = 64
_NT = 16
_NC = 2
_NW = _NC * _NT
_CH = 128
_RPT = 632
_NP = _NT * _RPT
_EALIGN = _NW * _CH

@functools.cache
def _mesh():
    return plsc.VectorSubcoreMesh(core_axis_name="c", subcore_axis_name="s",
                                  num_cores=_NC, num_subcores=_NT)


def _c16(j):
    return jnp.full((16,), j, dtype=jnp.int32)


_EPR = 128 // 32


_OFF = 16


def _scale_rows(ex_v, rows_v, srows_v, F):
    epr = 128 // F
    for j in range(_CH):
        sp = plsc.load_gather(ex_v, [_c16(_OFF + j)])
        for f0 in range(F // 16):
            srows_v[j // epr, pl.ds((j % epr) * F + 16 * f0, 16)] = (
                rows_v[j, pl.ds(16 * f0, 16)] * sp)


def _build_idx(dss_v, ri_v, F):
    epr = 128 // F
    io16 = lax.iota(jnp.int32, 16)
    for j in range(_CH):
        dsp = plsc.load_gather(dss_v, [_c16(_OFF + j)]) * F
        for f0 in range(F // 16):
            ri_v[j // epr, pl.ds((j % epr) * F + 16 * f0, 16)] = (
                dsp + (io16 + 16 * f0))


def _scatter_rows(srows_v, ri_v, acc_sh, sem, F):
    nq = (_CH * F) // 128
    for q in range(nq):
        pltpu.async_copy(srows_v.at[q], acc_sh.at[ri_v.at[q]], sem,
                         add=True).wait()


def _sc_diag(rows2, cols2, vals2, z1):
    EP2 = rows2.shape[0]
    nch = EP2 // (_NW * _CH)

    @functools.partial(
        pl.kernel,
        out_type=jax.ShapeDtypeStruct((_NC * _NP,), jnp.float32),
        mesh=_mesh(),
        compiler_params=pltpu.CompilerParams(needs_layout_passes=False, use_tc_tiling_on_sc=False),
        scratch_types=[
            pltpu.VMEM((_CH,), jnp.int32),
            pltpu.VMEM((_CH,), jnp.int32),
            pltpu.VMEM((_CH,), jnp.float32),
            pltpu.VMEM((_CH,), jnp.float32),
            pltpu.VMEM((_RPT,), jnp.float32),
            pltpu.VMEM_SHARED((_NP,), jnp.float32),
        ],
    )
    def k(rows_h, cols_h, vals_h, z1_h, d_out, r_v, c_v, v_v, w_v, st1_v, d_sh):
        cidx = lax.axis_index("c")
        sidx = lax.axis_index("s")
        wid = sidx * _NC + cidx
        rb = sidx * _RPT
        pltpu.sync_copy(z1_h, st1_v)
        pltpu.sync_copy(st1_v, d_sh.at[pl.ds(rb, _RPT)])
        plsc.subcore_barrier()

        def step(i, carry):
            base = (wid * nch + i) * _CH
            pltpu.sync_copy(rows_h.at[pl.ds(base, _CH)], r_v)
            pltpu.sync_copy(cols_h.at[pl.ds(base, _CH)], c_v)
            pltpu.sync_copy(vals_h.at[pl.ds(base, _CH)], v_v)
            for g in range(_CH // 16):
                r16 = r_v[pl.ds(16 * g, 16)]
                c16 = c_v[pl.ds(16 * g, 16)]
                v16 = v_v[pl.ds(16 * g, 16)]
                w_v[pl.ds(16 * g, 16)] = jnp.where(r16 == c16, v16, 0.0)
            pltpu.sync_copy(w_v, d_sh.at[r_v], add=True)
            return carry

        lax.fori_loop(0, nch, step, 0)
        plsc.subcore_barrier()
        pltpu.sync_copy(d_sh.at[pl.ds(rb, _RPT)], st1_v)
        pltpu.sync_copy(st1_v, d_out.at[pl.ds(cidx * _NP + rb, _RPT)])

    return k(rows2, cols2, vals2, z1)


def _sc_pv(rows2, cols2, vals2, dinv):
    EP2 = rows2.shape[0]
    nch = EP2 // (_NW * _CH)

    @functools.partial(
        pl.kernel,
        out_type=jax.ShapeDtypeStruct((EP2,), jnp.float32),
        mesh=_mesh(),
        compiler_params=pltpu.CompilerParams(needs_layout_passes=False, use_tc_tiling_on_sc=False),
        scratch_types=[
            pltpu.VMEM((_NP,), jnp.float32),
            pltpu.VMEM((_CH,), jnp.int32),
            pltpu.VMEM((_CH,), jnp.int32),
            pltpu.VMEM((_CH,), jnp.float32),
            pltpu.VMEM((_CH,), jnp.float32),
        ],
    )
    def k(rows_h, cols_h, vals_h, dinv_h, pv_out, dinv_v, r_v, c_v, v_v, w_v):
        cidx = lax.axis_index("c")
        sidx = lax.axis_index("s")
        wid = sidx * _NC + cidx
        pltpu.sync_copy(dinv_h, dinv_v)

        def step(i, carry):
            base = (wid * nch + i) * _CH
            pltpu.sync_copy(rows_h.at[pl.ds(base, _CH)], r_v)
            pltpu.sync_copy(cols_h.at[pl.ds(base, _CH)], c_v)
            pltpu.sync_copy(vals_h.at[pl.ds(base, _CH)], v_v)
            for g in range(_CH // 16):
                r16 = r_v[pl.ds(16 * g, 16)]
                c16 = c_v[pl.ds(16 * g, 16)]
                v16 = v_v[pl.ds(16 * g, 16)]
                dr = plsc.load_gather(dinv_v, [r16])
                dc = plsc.load_gather(dinv_v, [c16])
                w_v[pl.ds(16 * g, 16)] = dr * v16 * dc
            pltpu.sync_copy(w_v, pv_out.at[pl.ds(base, _CH)])
            return carry

        lax.fori_loop(0, nch, step, 0)

    return k(rows2, cols2, vals2, dinv)


def _sc_logits(lu_src, lu_dst, ld_src, ld_dst, rows2, cols2, pv, hs, hd, H,
               z2, z1, F):
    EPg = lu_src.shape[0]
    EP2 = rows2.shape[0]
    nchg = EPg // (_NW * _CH)
    nch2 = EP2 // (_NW * _CH)

    @functools.partial(
        pl.kernel,
        out_type=[
            jax.ShapeDtypeStruct((_NC * _NP,), jnp.float32),
            jax.ShapeDtypeStruct((_NC * _NP,), jnp.float32),
            jax.ShapeDtypeStruct((_NC * _NP * F,), jnp.float32),
        ],
        mesh=_mesh(),
        compiler_params=pltpu.CompilerParams(needs_layout_passes=False, use_tc_tiling_on_sc=False),
        scratch_types=[
            pltpu.VMEM((_NP,), jnp.float32),
            pltpu.VMEM((_NP,), jnp.float32),
            pltpu.VMEM((_CH,), jnp.int32),
            pltpu.VMEM((_CH,), jnp.int32),
            pltpu.VMEM((_CH + 16,), jnp.float32),
            pltpu.VMEM((_CH + 16,), jnp.int32),
            pltpu.VMEM((_CH, F), jnp.float32),
            pltpu.VMEM((_CH * F // 128, 128), jnp.float32),
            pltpu.VMEM((_CH * F // 128, 128), jnp.int32),
            pltpu.VMEM((_RPT,), jnp.float32),
            pltpu.VMEM((_RPT * F,), jnp.float32),
            pltpu.VMEM_SHARED((_NP,), jnp.float32),
            pltpu.VMEM_SHARED((_NP,), jnp.float32),
            pltpu.VMEM_SHARED((_NP * F,), jnp.float32),
            pltpu.SemaphoreType.DMA,
        ],
    )
    def k(lus_h, lud_h, lds_h, ldd_h, rows_h, cols_h, pv_h, hs_h, hd_h, h_h,
          z2_h, z1_h,
          s8u_o, s8d_o, accp_o,
          hs_v, hd_v, gi_v, si_v, ex_v, dss_v, rows_v, srows_v, ri_v,
          st1_v, st2_v, s8u_sh, s8d_sh, accp_sh, sem):
        cidx = lax.axis_index("c")
        sidx = lax.axis_index("s")
        wid = sidx * _NC + cidx
        rb = sidx * _RPT
        pltpu.sync_copy(z1_h, st1_v)
        pltpu.sync_copy(z2_h, st2_v)
        pltpu.sync_copy(st1_v, s8u_sh.at[pl.ds(rb, _RPT)])
        pltpu.sync_copy(st1_v, s8d_sh.at[pl.ds(rb, _RPT)])
        pltpu.sync_copy(st2_v, accp_sh.at[pl.ds(rb * F, _RPT * F)])
        pltpu.sync_copy(hs_h, hs_v)
        pltpu.sync_copy(hd_h, hd_v)
        plsc.subcore_barrier()

        def bound_pass(src_h, dst_h, s_sh):
            def step(i, carry):
                base = (wid * nchg + i) * _CH
                pltpu.sync_copy(src_h.at[pl.ds(base, _CH)], gi_v)
                pltpu.sync_copy(dst_h.at[pl.ds(base, _CH)], si_v)
                for g in range(_CH // 16):
                    s16 = gi_v[pl.ds(16 * g, 16)]
                    d16 = si_v[pl.ds(16 * g, 16)]
                    e = plsc.load_gather(hs_v, [s16]) + plsc.load_gather(hd_v, [d16])
                    e = jnp.where(e >= 0.0, e, 0.2 * e)
                    ex_v[pl.ds(_OFF + 16 * g, 16)] = jnp.exp(e * 0.125)
                pltpu.sync_copy(ex_v.at[pl.ds(_OFF, _CH)], s_sh.at[si_v],
                                add=True)
                return carry

            lax.fori_loop(0, nchg, step, 0)

        bound_pass(lus_h, lud_h, s8u_sh)
        bound_pass(lds_h, ldd_h, s8d_sh)

        def spmm_step(i, carry):
            base = (wid * nch2 + i) * _CH
            pltpu.sync_copy(cols_h.at[pl.ds(base, _CH)], gi_v)
            pltpu.sync_copy(rows_h.at[pl.ds(base, _CH)], si_v)
            pltpu.sync_copy(rows_h.at[pl.ds(base, _CH)], dss_v.at[pl.ds(_OFF, _CH)])
            pltpu.sync_copy(pv_h.at[pl.ds(base, _CH)], ex_v.at[pl.ds(_OFF, _CH)])
            pltpu.sync_copy(h_h.at[gi_v], rows_v)
            _scale_rows(ex_v, rows_v, srows_v, F)
            _build_idx(dss_v, ri_v, F)
            _scatter_rows(srows_v, ri_v, accp_sh, sem, F)
            return carry

        lax.fori_loop(0, nch2, spmm_step, 0)

        plsc.subcore_barrier()
        pltpu.sync_copy(s8u_sh.at[pl.ds(rb, _RPT)], st1_v)
        pltpu.sync_copy(st1_v, s8u_o.at[pl.ds(cidx * _NP + rb, _RPT)])
        pltpu.sync_copy(s8d_sh.at[pl.ds(rb, _RPT)], st1_v)
        pltpu.sync_copy(st1_v, s8d_o.at[pl.ds(cidx * _NP + rb, _RPT)])
        pltpu.sync_copy(accp_sh.at[pl.ds(rb * F, _RPT * F)], st2_v)
        pltpu.sync_copy(st2_v, accp_o.at[pl.ds((cidx * _NP + rb) * F, _RPT * F)])

    return k(lu_src, lu_dst, ld_src, ld_dst, rows2, cols2, pv, hs, hd, H,
             z2, z1)


def _sc_gat(lu_src, lu_dst, ld_src, ld_dst, hs, hd, m_u, m_d, G, z2, z1, F):
    EPg = lu_src.shape[0]
    nchg = EPg // (_NW * _CH)

    @functools.partial(
        pl.kernel,
        out_type=[
            jax.ShapeDtypeStruct((_NC * _NP * F,), jnp.float32),
            jax.ShapeDtypeStruct((_NC * _NP * F,), jnp.float32),
            jax.ShapeDtypeStruct((_NC * _NP,), jnp.float32),
            jax.ShapeDtypeStruct((_NC * _NP,), jnp.float32),
        ],
        mesh=_mesh(),
        compiler_params=pltpu.CompilerParams(needs_layout_passes=False, use_tc_tiling_on_sc=False),
        scratch_types=[
            pltpu.VMEM((_NP,), jnp.float32),
            pltpu.VMEM((_NP,), jnp.float32),
            pltpu.VMEM((_NP,), jnp.float32),
            pltpu.VMEM((_CH,), jnp.int32),
            pltpu.VMEM((_CH,), jnp.int32),
            pltpu.VMEM((_CH + 16,), jnp.float32),
            pltpu.VMEM((_CH + 16,), jnp.int32),
            pltpu.VMEM((_CH, F), jnp.float32),
            pltpu.VMEM((_CH * F // 128, 128), jnp.float32),
            pltpu.VMEM((_CH * F // 128, 128), jnp.int32),
            pltpu.VMEM((_RPT,), jnp.float32),
            pltpu.VMEM((_RPT * F,), jnp.float32),
            pltpu.VMEM_SHARED((_NP * F,), jnp.float32),
            pltpu.VMEM_SHARED((_NP * F,), jnp.float32),
            pltpu.VMEM_SHARED((_NP,), jnp.float32),
            pltpu.VMEM_SHARED((_NP,), jnp.float32),
            pltpu.SemaphoreType.DMA,
        ],
    )
    def k(lus_h, lud_h, lds_h, ldd_h, hs_h, hd_h, mu_h, md_h, g_h, z2_h, z1_h,
          accu_o, accd_o, su_o, sd_o,
          hs_v, hd_v, m_v, gi_v, si_v, ex_v, dss_v, rows_v, srows_v, ri_v,
          st1_v, st2_v, accu_sh, accd_sh, su_sh, sd_sh, sem):
        cidx = lax.axis_index("c")
        sidx = lax.axis_index("s")
        wid = sidx * _NC + cidx
        rb = sidx * _RPT
        pltpu.sync_copy(z1_h, st1_v)
        pltpu.sync_copy(z2_h, st2_v)
        pltpu.sync_copy(st2_v, accu_sh.at[pl.ds(rb * F, _RPT * F)])
        pltpu.sync_copy(st2_v, accd_sh.at[pl.ds(rb * F, _RPT * F)])
        pltpu.sync_copy(st1_v, su_sh.at[pl.ds(rb, _RPT)])
        pltpu.sync_copy(st1_v, sd_sh.at[pl.ds(rb, _RPT)])
        pltpu.sync_copy(hs_h, hs_v)
        pltpu.sync_copy(hd_h, hd_v)
        plsc.subcore_barrier()

        def gat_pass(src_h, dst_h, acc_sh, s_sh):
            def step(i, carry):
                base = (wid * nchg + i) * _CH
                pltpu.sync_copy(src_h.at[pl.ds(base, _CH)], gi_v)
                pltpu.sync_copy(dst_h.at[pl.ds(base, _CH)], si_v)
                pltpu.sync_copy(dst_h.at[pl.ds(base, _CH)], dss_v.at[pl.ds(_OFF, _CH)])
                pltpu.sync_copy(g_h.at[gi_v], rows_v)
                for g in range(_CH // 16):
                    s16 = gi_v[pl.ds(16 * g, 16)]
                    d16 = si_v[pl.ds(16 * g, 16)]
                    e = plsc.load_gather(hs_v, [s16]) + plsc.load_gather(hd_v, [d16])
                    e = jnp.where(e >= 0.0, e, 0.2 * e)
                    ex_v[pl.ds(_OFF + 16 * g, 16)] = jnp.exp(e - plsc.load_gather(m_v, [d16]))
                _scale_rows(ex_v, rows_v, srows_v, F)
                _build_idx(dss_v, ri_v, F)
                pltpu.sync_copy(ex_v.at[pl.ds(_OFF, _CH)], s_sh.at[si_v],
                                add=True)
                _scatter_rows(srows_v, ri_v, acc_sh, sem, F)
                return carry

            lax.fori_loop(0, nchg, step, 0)

        pltpu.sync_copy(mu_h, m_v)
        gat_pass(lus_h, lud_h, accu_sh, su_sh)
        pltpu.sync_copy(md_h, m_v)
        gat_pass(lds_h, ldd_h, accd_sh, sd_sh)

        plsc.subcore_barrier()
        pltpu.sync_copy(accu_sh.at[pl.ds(rb * F, _RPT * F)], st2_v)
        pltpu.sync_copy(st2_v, accu_o.at[pl.ds((cidx * _NP + rb) * F, _RPT * F)])
        pltpu.sync_copy(accd_sh.at[pl.ds(rb * F, _RPT * F)], st2_v)
        pltpu.sync_copy(st2_v, accd_o.at[pl.ds((cidx * _NP + rb) * F, _RPT * F)])
        pltpu.sync_copy(su_sh.at[pl.ds(rb, _RPT)], st1_v)
        pltpu.sync_copy(st1_v, su_o.at[pl.ds(cidx * _NP + rb, _RPT)])
        pltpu.sync_copy(sd_sh.at[pl.ds(rb, _RPT)], st1_v)
        pltpu.sync_copy(st1_v, sd_o.at[pl.ds(cidx * _NP + rb, _RPT)])

    return k(lu_src, lu_dst, ld_src, ld_dst, hs, hd, m_u, m_d, G, z2, z1)


_BM = 632


def _tc_dinv(d_parts):
    def body(d_ref, o_ref):
        d = d_ref[0] + d_ref[1]
        o_ref[...] = lax.rsqrt(jnp.where(d > 0.0, d, 1.0))

    return pl.pallas_call(
        body,
        out_shape=jax.ShapeDtypeStruct((_NP, 1), jnp.float32),
    )(d_parts)


def _tc_m(s8u, s8d):
    def body(u_ref, d_ref, mu_ref, md_ref):
        for r, o in ((u_ref, mu_ref), (d_ref, md_ref)):
            s8 = r[0] + r[1]
            o[...] = jnp.where(s8 > 0.0, 8.0 * jnp.log(jnp.maximum(s8, 1e-30)),
                               0.0)

    return pl.pallas_call(
        body,
        out_shape=[
            jax.ShapeDtypeStruct((_NP, 1), jnp.float32),
            jax.ShapeDtypeStruct((_NP, 1), jnp.float32),
        ],
    )(s8u, s8d)


def _tc_proj(x, Wp, Wg, a_s, a_d):
    Din = x.shape[1]
    F = Wp.shape[1]
    grid = (_NP // _BM,)

    def body(x_ref, wp_ref, wg_ref, as_ref, ad_ref, H_ref, G_ref, hs_ref, hd_ref):
        x_ = x_ref[...]
        H = jnp.dot(x_, wp_ref[...], preferred_element_type=jnp.float32)
        G = jnp.dot(x_, wg_ref[...], preferred_element_type=jnp.float32)
        H_ref[...] = H
        G_ref[...] = G
        hs_ref[...] = jnp.sum(G * as_ref[...], axis=1, keepdims=True)
        hd_ref[...] = jnp.sum(G * ad_ref[...], axis=1, keepdims=True)

    return pl.pallas_call(
        body,
        grid=grid,
        in_specs=[
            pl.BlockSpec((_BM, Din), lambda i: (i, 0)),
            pl.BlockSpec((Din, F), lambda i: (0, 0)),
            pl.BlockSpec((Din, F), lambda i: (0, 0)),
            pl.BlockSpec((1, F), lambda i: (0, 0)),
            pl.BlockSpec((1, F), lambda i: (0, 0)),
        ],
        out_specs=[
            pl.BlockSpec((_BM, F), lambda i: (i, 0)),
            pl.BlockSpec((_BM, F), lambda i: (i, 0)),
            pl.BlockSpec((_BM, 1), lambda i: (i, 0)),
            pl.BlockSpec((_BM, 1), lambda i: (i, 0)),
        ],
        out_shape=[
            jax.ShapeDtypeStruct((_NP, F), jnp.float32),
            jax.ShapeDtypeStruct((_NP, F), jnp.float32),
            jax.ShapeDtypeStruct((_NP, 1), jnp.float32),
            jax.ShapeDtypeStruct((_NP, 1), jnp.float32),
        ],
    )(x, Wp, Wg, a_s, a_d)


def _tc_merge(accu, accd, accp, su, sd):
    F = accu.shape[2]
    grid = (_NP // _BM,)

    def body(pu_ref, pd_ref, pp_ref, su_ref, sd_ref, o_ref):
        hu = pu_ref[0] + pu_ref[1]
        hd_ = pd_ref[0] + pd_ref[1]
        hp = pp_ref[0] + pp_ref[1]
        su_ = su_ref[0] + su_ref[1] + 1e-16
        sd_ = sd_ref[0] + sd_ref[1] + 1e-16
        o_ref[...] = jnp.maximum(hu / su_ + hd_ / sd_ + hp, 0.0)

    return pl.pallas_call(
        body,
        grid=grid,
        in_specs=[
            pl.BlockSpec((2, _BM, F), lambda i: (0, i, 0)),
            pl.BlockSpec((2, _BM, F), lambda i: (0, i, 0)),
            pl.BlockSpec((2, _BM, F), lambda i: (0, i, 0)),
            pl.BlockSpec((2, _BM, 1), lambda i: (0, i, 0)),
            pl.BlockSpec((2, _BM, 1), lambda i: (0, i, 0)),
        ],
        out_specs=pl.BlockSpec((_BM, F), lambda i: (i, 0)),
        out_shape=jax.ShapeDtypeStruct((_NP, F), jnp.float32),
    )(accu, accd, accp, su, sd)


def _tc_final(accu, accd, accp, su, sd, bids):
    F = accu.shape[2]
    grid = (_NP // _BM,)
    nsteps = _NP // _BM

    def body(pu_ref, pd_ref, pp_ref, su_ref, sd_ref, b_ref, o_ref, sums, cnt):
        i = pl.program_id(0)

        @pl.when(i == 0)
        def _():
            sums[...] = jnp.zeros_like(sums)
            cnt[...] = jnp.zeros_like(cnt)

        hu = pu_ref[0] + pu_ref[1]
        hd_ = pd_ref[0] + pd_ref[1]
        hp = pp_ref[0] + pp_ref[1]
        su_ = su_ref[0] + su_ref[1] + 1e-16
        sd_ = sd_ref[0] + sd_ref[1] + 1e-16
        x4 = jnp.maximum(hu / su_ + hd_ / sd_ + hp, 0.0)
        rowid = i * _BM + lax.broadcasted_iota(jnp.int32, (_BM, 1), 0)
        valid = rowid < _N
        cols = lax.broadcasted_iota(jnp.int32, (_BM, _NB), 1)
        oh = jnp.where((b_ref[...] == cols) & valid, 1.0, 0.0)
        sums[...] += lax.dot_general(oh, x4, (((0,), (0,)), ((), ())),
                                     preferred_element_type=jnp.float32)
        cnt[...] += lax.dot_general(oh, jnp.ones((_BM, 1), jnp.float32),
                                    (((0,), (0,)), ((), ())),
                                    preferred_element_type=jnp.float32)

        @pl.when(i == nsteps - 1)
        def _():
            pooled = sums[...] / jnp.maximum(cnt[...], 1.0)
            cmask = lax.broadcasted_iota(jnp.int32, (_NB, F), 1) < 10
            mx = jnp.max(jnp.where(cmask, pooled, -jnp.inf), axis=1,
                         keepdims=True)
            ex = jnp.where(cmask, jnp.exp(pooled - mx), 0.0)
            o_ref[...] = ex / jnp.sum(ex, axis=1, keepdims=True)

    return pl.pallas_call(
        body,
        grid=grid,
        in_specs=[
            pl.BlockSpec((2, _BM, F), lambda i: (0, i, 0)),
            pl.BlockSpec((2, _BM, F), lambda i: (0, i, 0)),
            pl.BlockSpec((2, _BM, F), lambda i: (0, i, 0)),
            pl.BlockSpec((2, _BM, 1), lambda i: (0, i, 0)),
            pl.BlockSpec((2, _BM, 1), lambda i: (0, i, 0)),
            pl.BlockSpec((_BM, 1), lambda i: (i, 0)),
        ],
        out_specs=pl.BlockSpec((_NB, F), lambda i: (0, 0)),
        out_shape=jax.ShapeDtypeStruct((_NB, F), jnp.float32),
        scratch_shapes=[
            pltpu.VMEM((_NB, F), jnp.float32),
            pltpu.VMEM((_NB, 1), jnp.float32),
        ],
    )(accu, accd, accp, su, sd, bids)


def _pad_edges(arrs, n, target):
    padn = target - n
    pidx = (jnp.arange(padn, dtype=jnp.int32) % 16) + _N
    out = []
    for a, kind in arrs:
        if kind == "idx":
            out.append(jnp.concatenate([a, pidx]))
        else:
            out.append(jnp.concatenate([a, jnp.zeros((padn,), a.dtype)]))
    return out


def kernel(x1, lu_idx, lu_vals, ld_idx, ld_vals, batch1,
           Wp1, Wg1, asrc1, adst1, Wp2, Wg2, asrc2, adst2,
           Wp3, Wg3, asrc3, adst3, Wp4, Wg4, asrc4, adst4):
    E = lu_idx.shape[1]
    EPg = -(-E // _EALIGN) * _EALIGN
    E2 = 2 * E
    EP2 = -(-E2 // _EALIGN) * _EALIGN

    lu_src, lu_dst = _pad_edges([(lu_idx[0], "idx"), (lu_idx[1], "idx")], E, EPg)
    ld_src, ld_dst = _pad_edges([(ld_idx[0], "idx"), (ld_idx[1], "idx")], E, EPg)
    rows2, cols2, vals2 = _pad_edges(
        [(jnp.concatenate([lu_idx[0], ld_idx[0]]), "idx"),
         (jnp.concatenate([lu_idx[1], ld_idx[1]]), "idx"),
         (jnp.concatenate([lu_vals, ld_vals]), "val")], E2, EP2)

    xp = jnp.pad(x1, ((0, _NP - _N), (0, 0)))
    bids = jnp.pad(batch1, (0, _NP - _N)).reshape(_NP, 1)
    z1 = jnp.zeros((_RPT,), jnp.float32)

    d_parts = _sc_diag(rows2, cols2, vals2, z1)
    dinv = _tc_dinv(d_parts.reshape(_NC, _NP, 1))
    pv = _sc_pv(rows2, cols2, vals2, dinv.reshape(_NP))

    Wp4p = jnp.pad(Wp4, ((0, 0), (0, 6)))
    Wg4p = jnp.pad(Wg4, ((0, 0), (0, 6)))
    asrc4p = jnp.pad(asrc4, (0, 6))
    adst4p = jnp.pad(adst4, (0, 6))

    layers = [
        (Wp1, Wg1, asrc1, adst1, 32),
        (Wp2, Wg2, asrc2, adst2, 32),
        (Wp3, Wg3, asrc3, adst3, 32),
        (Wp4p, Wg4p, asrc4p, adst4p, 16),
    ]

    x = xp
    parts = None
    for li, (Wp, Wg, a_s, a_d, F) in enumerate(layers):
        if parts is not None:
            x = _tc_merge(*parts)
        H, G, hs, hd = _tc_proj(x, Wp, Wg, a_s.reshape(1, -1), a_d.reshape(1, -1))
        z2 = jnp.zeros((_RPT * F,), jnp.float32)
        hs1 = hs.reshape(_NP)
        hd1 = hd.reshape(_NP)
        s8u, s8d, accp = _sc_logits(
            lu_src, lu_dst, ld_src, ld_dst, rows2, cols2, pv,
            hs1, hd1, H, z2, z1, F)
        m_u, m_d = _tc_m(s8u.reshape(_NC, _NP, 1), s8d.reshape(_NC, _NP, 1))
        accu, accd, su, sd = _sc_gat(
            lu_src, lu_dst, ld_src, ld_dst, hs1, hd1,
            m_u.reshape(_NP), m_d.reshape(_NP), G, z2, z1, F)
        parts = (accu.reshape(_NC, _NP, F), accd.reshape(_NC, _NP, F),
                 accp.reshape(_NC, _NP, F), su.reshape(_NC, _NP, 1),
                 sd.reshape(_NC, _NP, 1))

    out = _tc_final(*parts, bids)
    return out[:, :10]


# speedup vs baseline: 11.8571x; 1.2170x over previous
import functools

import jax
import jax.numpy as jnp
from jax import lax
from jax.experimental import pallas as pl
from jax.experimental.pallas import tpu as pltpu
from jax.experimental.pallas import tpu_sc as plsc

_N = 10000
_NB = 64
_NT = 16
_NC = 2
_NW = _NC * _NT
_CH = 128
_RPT = 632
_NP = _NT * _RPT
_EALIGN = _NW * _CH

@functools.cache
def _mesh():
    return plsc.VectorSubcoreMesh(core_axis_name="c", subcore_axis_name="s",
                                  num_cores=_NC, num_subcores=_NT)


def _c16(j):
    return jnp.full((16,), j, dtype=jnp.int32)


_EPR = 128 // 32


_OFF = 16


def _scale_rows(ex_v, rows_v, srows_v, F):
    epr = 128 // F
    for j in range(_CH):
        sp = plsc.load_gather(ex_v, [_c16(_OFF + j)])
        for f0 in range(F // 16):
            srows_v[j // epr, pl.ds((j % epr) * F + 16 * f0, 16)] = (
                rows_v[j, pl.ds(16 * f0, 16)] * sp)


def _build_idx(dss_v, ri_v, F):
    epr = 128 // F
    io16 = lax.iota(jnp.int32, 16)
    for j in range(_CH):
        dsp = plsc.load_gather(dss_v, [_c16(_OFF + j)]) * F
        for f0 in range(F // 16):
            ri_v[j // epr, pl.ds((j % epr) * F + 16 * f0, 16)] = (
                dsp + (io16 + 16 * f0))


def _scatter_rows(srows_v, ri_v, acc_sh, sem, F):
    nq = (_CH * F) // 128
    cps = [pltpu.async_copy(srows_v.at[q], acc_sh.at[ri_v.at[q]], sem,
                            add=True) for q in range(nq)]
    for c in cps:
        c.wait()


def _sc_diag(rows2, cols2, vals2, z1):
    EP2 = rows2.shape[0]
    nch = EP2 // (_NW * _CH)

    @functools.partial(
        pl.kernel,
        out_type=jax.ShapeDtypeStruct((_NC * _NP,), jnp.float32),
        mesh=_mesh(),
        compiler_params=pltpu.CompilerParams(needs_layout_passes=False, use_tc_tiling_on_sc=False),
        scratch_types=[
            pltpu.VMEM((_CH,), jnp.int32),
            pltpu.VMEM((_CH,), jnp.int32),
            pltpu.VMEM((_CH,), jnp.float32),
            pltpu.VMEM((_CH,), jnp.float32),
            pltpu.VMEM((_RPT,), jnp.float32),
            pltpu.VMEM_SHARED((_NP,), jnp.float32),
        ],
    )
    def k(rows_h, cols_h, vals_h, z1_h, d_out, r_v, c_v, v_v, w_v, st1_v, d_sh):
        cidx = lax.axis_index("c")
        sidx = lax.axis_index("s")
        wid = sidx * _NC + cidx
        rb = sidx * _RPT
        pltpu.sync_copy(z1_h, st1_v)
        pltpu.sync_copy(st1_v, d_sh.at[pl.ds(rb, _RPT)])
        plsc.subcore_barrier()

        def step(i, carry):
            base = (wid * nch + i) * _CH
            pltpu.sync_copy(rows_h.at[pl.ds(base, _CH)], r_v)
            pltpu.sync_copy(cols_h.at[pl.ds(base, _CH)], c_v)
            pltpu.sync_copy(vals_h.at[pl.ds(base, _CH)], v_v)
            for g in range(_CH // 16):
                r16 = r_v[pl.ds(16 * g, 16)]
                c16 = c_v[pl.ds(16 * g, 16)]
                v16 = v_v[pl.ds(16 * g, 16)]
                w_v[pl.ds(16 * g, 16)] = jnp.where(r16 == c16, v16, 0.0)
            pltpu.sync_copy(w_v, d_sh.at[r_v], add=True)
            return carry

        lax.fori_loop(0, nch, step, 0)
        plsc.subcore_barrier()
        pltpu.sync_copy(d_sh.at[pl.ds(rb, _RPT)], st1_v)
        pltpu.sync_copy(st1_v, d_out.at[pl.ds(cidx * _NP + rb, _RPT)])

    return k(rows2, cols2, vals2, z1)


def _sc_pv(rows2, cols2, vals2, dinv):
    EP2 = rows2.shape[0]
    nch = EP2 // (_NW * _CH)

    @functools.partial(
        pl.kernel,
        out_type=jax.ShapeDtypeStruct((EP2,), jnp.float32),
        mesh=_mesh(),
        compiler_params=pltpu.CompilerParams(needs_layout_passes=False, use_tc_tiling_on_sc=False),
        scratch_types=[
            pltpu.VMEM((_NP,), jnp.float32),
            pltpu.VMEM((_CH,), jnp.int32),
            pltpu.VMEM((_CH,), jnp.int32),
            pltpu.VMEM((_CH,), jnp.float32),
            pltpu.VMEM((_CH,), jnp.float32),
        ],
    )
    def k(rows_h, cols_h, vals_h, dinv_h, pv_out, dinv_v, r_v, c_v, v_v, w_v):
        cidx = lax.axis_index("c")
        sidx = lax.axis_index("s")
        wid = sidx * _NC + cidx
        pltpu.sync_copy(dinv_h, dinv_v)

        def step(i, carry):
            base = (wid * nch + i) * _CH
            pltpu.sync_copy(rows_h.at[pl.ds(base, _CH)], r_v)
            pltpu.sync_copy(cols_h.at[pl.ds(base, _CH)], c_v)
            pltpu.sync_copy(vals_h.at[pl.ds(base, _CH)], v_v)
            for g in range(_CH // 16):
                r16 = r_v[pl.ds(16 * g, 16)]
                c16 = c_v[pl.ds(16 * g, 16)]
                v16 = v_v[pl.ds(16 * g, 16)]
                dr = plsc.load_gather(dinv_v, [r16])
                dc = plsc.load_gather(dinv_v, [c16])
                w_v[pl.ds(16 * g, 16)] = dr * v16 * dc
            pltpu.sync_copy(w_v, pv_out.at[pl.ds(base, _CH)])
            return carry

        lax.fori_loop(0, nch, step, 0)

    return k(rows2, cols2, vals2, dinv)


def _sc_logits(lu_src, lu_dst, ld_src, ld_dst, rows2, cols2, pv, hs, hd, H,
               z2, z1, F):
    EPg = lu_src.shape[0]
    EP2 = rows2.shape[0]
    nchg = EPg // (_NW * _CH)
    nch2 = EP2 // (_NW * _CH)

    @functools.partial(
        pl.kernel,
        out_type=[
            jax.ShapeDtypeStruct((_NC * _NP,), jnp.float32),
            jax.ShapeDtypeStruct((_NC * _NP,), jnp.float32),
            jax.ShapeDtypeStruct((_NC * _NP * F,), jnp.float32),
        ],
        mesh=_mesh(),
        compiler_params=pltpu.CompilerParams(needs_layout_passes=False, use_tc_tiling_on_sc=False),
        scratch_types=[
            pltpu.VMEM((_NP,), jnp.float32),
            pltpu.VMEM((_NP,), jnp.float32),
            pltpu.VMEM((_CH,), jnp.int32),
            pltpu.VMEM((_CH,), jnp.int32),
            pltpu.VMEM((_CH + 16,), jnp.float32),
            pltpu.VMEM((_CH + 16,), jnp.int32),
            pltpu.VMEM((_CH, F), jnp.float32),
            pltpu.VMEM((_CH * F // 128, 128), jnp.float32),
            pltpu.VMEM((_CH * F // 128, 128), jnp.int32),
            pltpu.VMEM((_RPT,), jnp.float32),
            pltpu.VMEM((_RPT * F,), jnp.float32),
            pltpu.VMEM_SHARED((_NP,), jnp.float32),
            pltpu.VMEM_SHARED((_NP,), jnp.float32),
            pltpu.VMEM_SHARED((_NP * F,), jnp.float32),
            pltpu.SemaphoreType.DMA,
        ],
    )
    def k(lus_h, lud_h, lds_h, ldd_h, rows_h, cols_h, pv_h, hs_h, hd_h, h_h,
          z2_h, z1_h,
          s8u_o, s8d_o, accp_o,
          hs_v, hd_v, gi_v, si_v, ex_v, dss_v, rows_v, srows_v, ri_v,
          st1_v, st2_v, s8u_sh, s8d_sh, accp_sh, sem):
        cidx = lax.axis_index("c")
        sidx = lax.axis_index("s")
        wid = sidx * _NC + cidx
        rb = sidx * _RPT
        pltpu.sync_copy(z1_h, st1_v)
        pltpu.sync_copy(z2_h, st2_v)
        pltpu.sync_copy(st1_v, s8u_sh.at[pl.ds(rb, _RPT)])
        pltpu.sync_copy(st1_v, s8d_sh.at[pl.ds(rb, _RPT)])
        pltpu.sync_copy(st2_v, accp_sh.at[pl.ds(rb * F, _RPT * F)])
        pltpu.sync_copy(hs_h, hs_v)
        pltpu.sync_copy(hd_h, hd_v)
        plsc.subcore_barrier()

        def bound_pass(src_h, dst_h, s_sh):
            def step(i, carry):
                base = (wid * nchg + i) * _CH
                pltpu.sync_copy(src_h.at[pl.ds(base, _CH)], gi_v)
                pltpu.sync_copy(dst_h.at[pl.ds(base, _CH)], si_v)
                for g in range(_CH // 16):
                    s16 = gi_v[pl.ds(16 * g, 16)]
                    d16 = si_v[pl.ds(16 * g, 16)]
                    e = plsc.load_gather(hs_v, [s16]) + plsc.load_gather(hd_v, [d16])
                    e = jnp.where(e >= 0.0, e, 0.2 * e)
                    ex_v[pl.ds(_OFF + 16 * g, 16)] = jnp.exp(e * 0.125)
                pltpu.sync_copy(ex_v.at[pl.ds(_OFF, _CH)], s_sh.at[si_v],
                                add=True)
                return carry

            lax.fori_loop(0, nchg, step, 0)

        bound_pass(lus_h, lud_h, s8u_sh)
        bound_pass(lds_h, ldd_h, s8d_sh)

        def spmm_step(i, carry):
            base = (wid * nch2 + i) * _CH
            pltpu.sync_copy(cols_h.at[pl.ds(base, _CH)], gi_v)
            pltpu.sync_copy(rows_h.at[pl.ds(base, _CH)], si_v)
            pltpu.sync_copy(rows_h.at[pl.ds(base, _CH)], dss_v.at[pl.ds(_OFF, _CH)])
            pltpu.sync_copy(pv_h.at[pl.ds(base, _CH)], ex_v.at[pl.ds(_OFF, _CH)])
            pltpu.sync_copy(h_h.at[gi_v], rows_v)
            _scale_rows(ex_v, rows_v, srows_v, F)
            _build_idx(dss_v, ri_v, F)
            _scatter_rows(srows_v, ri_v, accp_sh, sem, F)
            return carry

        lax.fori_loop(0, nch2, spmm_step, 0)

        plsc.subcore_barrier()
        pltpu.sync_copy(s8u_sh.at[pl.ds(rb, _RPT)], st1_v)
        pltpu.sync_copy(st1_v, s8u_o.at[pl.ds(cidx * _NP + rb, _RPT)])
        pltpu.sync_copy(s8d_sh.at[pl.ds(rb, _RPT)], st1_v)
        pltpu.sync_copy(st1_v, s8d_o.at[pl.ds(cidx * _NP + rb, _RPT)])
        pltpu.sync_copy(accp_sh.at[pl.ds(rb * F, _RPT * F)], st2_v)
        pltpu.sync_copy(st2_v, accp_o.at[pl.ds((cidx * _NP + rb) * F, _RPT * F)])

    return k(lu_src, lu_dst, ld_src, ld_dst, rows2, cols2, pv, hs, hd, H,
             z2, z1)


def _sc_gat(lu_src, lu_dst, ld_src, ld_dst, hs, hd, m_u, m_d, G, z2, z1, F):
    EPg = lu_src.shape[0]
    nchg = EPg // (_NW * _CH)

    @functools.partial(
        pl.kernel,
        out_type=[
            jax.ShapeDtypeStruct((_NC * _NP * F,), jnp.float32),
            jax.ShapeDtypeStruct((_NC * _NP * F,), jnp.float32),
            jax.ShapeDtypeStruct((_NC * _NP,), jnp.float32),
            jax.ShapeDtypeStruct((_NC * _NP,), jnp.float32),
        ],
        mesh=_mesh(),
        compiler_params=pltpu.CompilerParams(needs_layout_passes=False, use_tc_tiling_on_sc=False),
        scratch_types=[
            pltpu.VMEM((_NP,), jnp.float32),
            pltpu.VMEM((_NP,), jnp.float32),
            pltpu.VMEM((_NP,), jnp.float32),
            pltpu.VMEM((_CH,), jnp.int32),
            pltpu.VMEM((_CH,), jnp.int32),
            pltpu.VMEM((_CH + 16,), jnp.float32),
            pltpu.VMEM((_CH + 16,), jnp.int32),
            pltpu.VMEM((_CH, F), jnp.float32),
            pltpu.VMEM((_CH * F // 128, 128), jnp.float32),
            pltpu.VMEM((_CH * F // 128, 128), jnp.int32),
            pltpu.VMEM((_RPT,), jnp.float32),
            pltpu.VMEM((_RPT * F,), jnp.float32),
            pltpu.VMEM_SHARED((_NP * F,), jnp.float32),
            pltpu.VMEM_SHARED((_NP * F,), jnp.float32),
            pltpu.VMEM_SHARED((_NP,), jnp.float32),
            pltpu.VMEM_SHARED((_NP,), jnp.float32),
            pltpu.SemaphoreType.DMA,
        ],
    )
    def k(lus_h, lud_h, lds_h, ldd_h, hs_h, hd_h, mu_h, md_h, g_h, z2_h, z1_h,
          accu_o, accd_o, su_o, sd_o,
          hs_v, hd_v, m_v, gi_v, si_v, ex_v, dss_v, rows_v, srows_v, ri_v,
          st1_v, st2_v, accu_sh, accd_sh, su_sh, sd_sh, sem):
        cidx = lax.axis_index("c")
        sidx = lax.axis_index("s")
        wid = sidx * _NC + cidx
        rb = sidx * _RPT
        pltpu.sync_copy(z1_h, st1_v)
        pltpu.sync_copy(z2_h, st2_v)
        pltpu.sync_copy(st2_v, accu_sh.at[pl.ds(rb * F, _RPT * F)])
        pltpu.sync_copy(st2_v, accd_sh.at[pl.ds(rb * F, _RPT * F)])
        pltpu.sync_copy(st1_v, su_sh.at[pl.ds(rb, _RPT)])
        pltpu.sync_copy(st1_v, sd_sh.at[pl.ds(rb, _RPT)])
        pltpu.sync_copy(hs_h, hs_v)
        pltpu.sync_copy(hd_h, hd_v)
        plsc.subcore_barrier()

        def gat_pass(src_h, dst_h, acc_sh, s_sh):
            def step(i, carry):
                base = (wid * nchg + i) * _CH
                pltpu.sync_copy(src_h.at[pl.ds(base, _CH)], gi_v)
                pltpu.sync_copy(dst_h.at[pl.ds(base, _CH)], si_v)
                pltpu.sync_copy(dst_h.at[pl.ds(base, _CH)], dss_v.at[pl.ds(_OFF, _CH)])
                pltpu.sync_copy(g_h.at[gi_v], rows_v)
                for g in range(_CH // 16):
                    s16 = gi_v[pl.ds(16 * g, 16)]
                    d16 = si_v[pl.ds(16 * g, 16)]
                    e = plsc.load_gather(hs_v, [s16]) + plsc.load_gather(hd_v, [d16])
                    e = jnp.where(e >= 0.0, e, 0.2 * e)
                    ex_v[pl.ds(_OFF + 16 * g, 16)] = jnp.exp(e - plsc.load_gather(m_v, [d16]))
                _scale_rows(ex_v, rows_v, srows_v, F)
                _build_idx(dss_v, ri_v, F)
                pltpu.sync_copy(ex_v.at[pl.ds(_OFF, _CH)], s_sh.at[si_v],
                                add=True)
                _scatter_rows(srows_v, ri_v, acc_sh, sem, F)
                return carry

            lax.fori_loop(0, nchg, step, 0)

        pltpu.sync_copy(mu_h, m_v)
        gat_pass(lus_h, lud_h, accu_sh, su_sh)
        pltpu.sync_copy(md_h, m_v)
        gat_pass(lds_h, ldd_h, accd_sh, sd_sh)

        plsc.subcore_barrier()
        pltpu.sync_copy(accu_sh.at[pl.ds(rb * F, _RPT * F)], st2_v)
        pltpu.sync_copy(st2_v, accu_o.at[pl.ds((cidx * _NP + rb) * F, _RPT * F)])
        pltpu.sync_copy(accd_sh.at[pl.ds(rb * F, _RPT * F)], st2_v)
        pltpu.sync_copy(st2_v, accd_o.at[pl.ds((cidx * _NP + rb) * F, _RPT * F)])
        pltpu.sync_copy(su_sh.at[pl.ds(rb, _RPT)], st1_v)
        pltpu.sync_copy(st1_v, su_o.at[pl.ds(cidx * _NP + rb, _RPT)])
        pltpu.sync_copy(sd_sh.at[pl.ds(rb, _RPT)], st1_v)
        pltpu.sync_copy(st1_v, sd_o.at[pl.ds(cidx * _NP + rb, _RPT)])

    return k(lu_src, lu_dst, ld_src, ld_dst, hs, hd, m_u, m_d, G, z2, z1)


_BM = 632


def _tc_dinv(d_parts):
    def body(d_ref, o_ref):
        d = d_ref[0] + d_ref[1]
        o_ref[...] = lax.rsqrt(jnp.where(d > 0.0, d, 1.0))

    return pl.pallas_call(
        body,
        out_shape=jax.ShapeDtypeStruct((_NP, 1), jnp.float32),
    )(d_parts)


def _tc_m(s8u, s8d):
    def body(u_ref, d_ref, mu_ref, md_ref):
        for r, o in ((u_ref, mu_ref), (d_ref, md_ref)):
            s8 = r[0] + r[1]
            o[...] = jnp.where(s8 > 0.0, 8.0 * jnp.log(jnp.maximum(s8, 1e-30)),
                               0.0)

    return pl.pallas_call(
        body,
        out_shape=[
            jax.ShapeDtypeStruct((_NP, 1), jnp.float32),
            jax.ShapeDtypeStruct((_NP, 1), jnp.float32),
        ],
    )(s8u, s8d)


def _tc_proj(x, Wp, Wg, a_s, a_d):
    Din = x.shape[1]
    F = Wp.shape[1]
    grid = (_NP // _BM,)

    def body(x_ref, wp_ref, wg_ref, as_ref, ad_ref, H_ref, G_ref, hs_ref, hd_ref):
        x_ = x_ref[...]
        H = jnp.dot(x_, wp_ref[...], preferred_element_type=jnp.float32)
        G = jnp.dot(x_, wg_ref[...], preferred_element_type=jnp.float32)
        H_ref[...] = H
        G_ref[...] = G
        hs_ref[...] = jnp.sum(G * as_ref[...], axis=1, keepdims=True)
        hd_ref[...] = jnp.sum(G * ad_ref[...], axis=1, keepdims=True)

    return pl.pallas_call(
        body,
        grid=grid,
        in_specs=[
            pl.BlockSpec((_BM, Din), lambda i: (i, 0)),
            pl.BlockSpec((Din, F), lambda i: (0, 0)),
            pl.BlockSpec((Din, F), lambda i: (0, 0)),
            pl.BlockSpec((1, F), lambda i: (0, 0)),
            pl.BlockSpec((1, F), lambda i: (0, 0)),
        ],
        out_specs=[
            pl.BlockSpec((_BM, F), lambda i: (i, 0)),
            pl.BlockSpec((_BM, F), lambda i: (i, 0)),
            pl.BlockSpec((_BM, 1), lambda i: (i, 0)),
            pl.BlockSpec((_BM, 1), lambda i: (i, 0)),
        ],
        out_shape=[
            jax.ShapeDtypeStruct((_NP, F), jnp.float32),
            jax.ShapeDtypeStruct((_NP, F), jnp.float32),
            jax.ShapeDtypeStruct((_NP, 1), jnp.float32),
            jax.ShapeDtypeStruct((_NP, 1), jnp.float32),
        ],
    )(x, Wp, Wg, a_s, a_d)


def _tc_merge(accu, accd, accp, su, sd):
    F = accu.shape[2]
    grid = (_NP // _BM,)

    def body(pu_ref, pd_ref, pp_ref, su_ref, sd_ref, o_ref):
        hu = pu_ref[0] + pu_ref[1]
        hd_ = pd_ref[0] + pd_ref[1]
        hp = pp_ref[0] + pp_ref[1]
        su_ = su_ref[0] + su_ref[1] + 1e-16
        sd_ = sd_ref[0] + sd_ref[1] + 1e-16
        o_ref[...] = jnp.maximum(hu / su_ + hd_ / sd_ + hp, 0.0)

    return pl.pallas_call(
        body,
        grid=grid,
        in_specs=[
            pl.BlockSpec((2, _BM, F), lambda i: (0, i, 0)),
            pl.BlockSpec((2, _BM, F), lambda i: (0, i, 0)),
            pl.BlockSpec((2, _BM, F), lambda i: (0, i, 0)),
            pl.BlockSpec((2, _BM, 1), lambda i: (0, i, 0)),
            pl.BlockSpec((2, _BM, 1), lambda i: (0, i, 0)),
        ],
        out_specs=pl.BlockSpec((_BM, F), lambda i: (i, 0)),
        out_shape=jax.ShapeDtypeStruct((_NP, F), jnp.float32),
    )(accu, accd, accp, su, sd)


def _tc_final(accu, accd, accp, su, sd, bids):
    F = accu.shape[2]
    grid = (_NP // _BM,)
    nsteps = _NP // _BM

    def body(pu_ref, pd_ref, pp_ref, su_ref, sd_ref, b_ref, o_ref, sums, cnt):
        i = pl.program_id(0)

        @pl.when(i == 0)
        def _():
            sums[...] = jnp.zeros_like(sums)
            cnt[...] = jnp.zeros_like(cnt)

        hu = pu_ref[0] + pu_ref[1]
        hd_ = pd_ref[0] + pd_ref[1]
        hp = pp_ref[0] + pp_ref[1]
        su_ = su_ref[0] + su_ref[1] + 1e-16
        sd_ = sd_ref[0] + sd_ref[1] + 1e-16
        x4 = jnp.maximum(hu / su_ + hd_ / sd_ + hp, 0.0)
        rowid = i * _BM + lax.broadcasted_iota(jnp.int32, (_BM, 1), 0)
        valid = rowid < _N
        cols = lax.broadcasted_iota(jnp.int32, (_BM, _NB), 1)
        oh = jnp.where((b_ref[...] == cols) & valid, 1.0, 0.0)
        sums[...] += lax.dot_general(oh, x4, (((0,), (0,)), ((), ())),
                                     preferred_element_type=jnp.float32)
        cnt[...] += lax.dot_general(oh, jnp.ones((_BM, 1), jnp.float32),
                                    (((0,), (0,)), ((), ())),
                                    preferred_element_type=jnp.float32)

        @pl.when(i == nsteps - 1)
        def _():
            pooled = sums[...] / jnp.maximum(cnt[...], 1.0)
            cmask = lax.broadcasted_iota(jnp.int32, (_NB, F), 1) < 10
            mx = jnp.max(jnp.where(cmask, pooled, -jnp.inf), axis=1,
                         keepdims=True)
            ex = jnp.where(cmask, jnp.exp(pooled - mx), 0.0)
            o_ref[...] = ex / jnp.sum(ex, axis=1, keepdims=True)

    return pl.pallas_call(
        body,
        grid=grid,
        in_specs=[
            pl.BlockSpec((2, _BM, F), lambda i: (0, i, 0)),
            pl.BlockSpec((2, _BM, F), lambda i: (0, i, 0)),
            pl.BlockSpec((2, _BM, F), lambda i: (0, i, 0)),
            pl.BlockSpec((2, _BM, 1), lambda i: (0, i, 0)),
            pl.BlockSpec((2, _BM, 1), lambda i: (0, i, 0)),
            pl.BlockSpec((_BM, 1), lambda i: (i, 0)),
        ],
        out_specs=pl.BlockSpec((_NB, F), lambda i: (0, 0)),
        out_shape=jax.ShapeDtypeStruct((_NB, F), jnp.float32),
        scratch_shapes=[
            pltpu.VMEM((_NB, F), jnp.float32),
            pltpu.VMEM((_NB, 1), jnp.float32),
        ],
    )(accu, accd, accp, su, sd, bids)


def _pad_edges(arrs, n, target):
    padn = target - n
    pidx = (jnp.arange(padn, dtype=jnp.int32) % 16) + _N
    out = []
    for a, kind in arrs:
        if kind == "idx":
            out.append(jnp.concatenate([a, pidx]))
        else:
            out.append(jnp.concatenate([a, jnp.zeros((padn,), a.dtype)]))
    return out


def kernel(x1, lu_idx, lu_vals, ld_idx, ld_vals, batch1,
           Wp1, Wg1, asrc1, adst1, Wp2, Wg2, asrc2, adst2,
           Wp3, Wg3, asrc3, adst3, Wp4, Wg4, asrc4, adst4):
    E = lu_idx.shape[1]
    EPg = -(-E // _EALIGN) * _EALIGN
    E2 = 2 * E
    EP2 = -(-E2 // _EALIGN) * _EALIGN

    lu_src, lu_dst = _pad_edges([(lu_idx[0], "idx"), (lu_idx[1], "idx")], E, EPg)
    ld_src, ld_dst = _pad_edges([(ld_idx[0], "idx"), (ld_idx[1], "idx")], E, EPg)
    rows2, cols2, vals2 = _pad_edges(
        [(jnp.concatenate([lu_idx[0], ld_idx[0]]), "idx"),
         (jnp.concatenate([lu_idx[1], ld_idx[1]]), "idx"),
         (jnp.concatenate([lu_vals, ld_vals]), "val")], E2, EP2)

    xp = jnp.pad(x1, ((0, _NP - _N), (0, 0)))
    bids = jnp.pad(batch1, (0, _NP - _N)).reshape(_NP, 1)
    z1 = jnp.zeros((_RPT,), jnp.float32)

    d_parts = _sc_diag(rows2, cols2, vals2, z1)
    dinv = _tc_dinv(d_parts.reshape(_NC, _NP, 1))
    pv = _sc_pv(rows2, cols2, vals2, dinv.reshape(_NP))

    Wp4p = jnp.pad(Wp4, ((0, 0), (0, 6)))
    Wg4p = jnp.pad(Wg4, ((0, 0), (0, 6)))
    asrc4p = jnp.pad(asrc4, (0, 6))
    adst4p = jnp.pad(adst4, (0, 6))

    layers = [
        (Wp1, Wg1, asrc1, adst1, 32),
        (Wp2, Wg2, asrc2, adst2, 32),
        (Wp3, Wg3, asrc3, adst3, 32),
        (Wp4p, Wg4p, asrc4p, adst4p, 16),
    ]

    x = xp
    parts = None
    for li, (Wp, Wg, a_s, a_d, F) in enumerate(layers):
        if parts is not None:
            x = _tc_merge(*parts)
        H, G, hs, hd = _tc_proj(x, Wp, Wg, a_s.reshape(1, -1), a_d.reshape(1, -1))
        z2 = jnp.zeros((_RPT * F,), jnp.float32)
        hs1 = hs.reshape(_NP)
        hd1 = hd.reshape(_NP)
        s8u, s8d, accp = _sc_logits(
            lu_src, lu_dst, ld_src, ld_dst, rows2, cols2, pv,
            hs1, hd1, H, z2, z1, F)
        m_u, m_d = _tc_m(s8u.reshape(_NC, _NP, 1), s8d.reshape(_NC, _NP, 1))
        accu, accd, su, sd = _sc_gat(
            lu_src, lu_dst, ld_src, ld_dst, hs1, hd1,
            m_u.reshape(_NP), m_d.reshape(_NP), G, z2, z1, F)
        parts = (accu.reshape(_NC, _NP, F), accd.reshape(_NC, _NP, F),
                 accp.reshape(_NC, _NP, F), su.reshape(_NC, _NP, 1),
                 sd.reshape(_NC, _NP, 1))

    out = _tc_final(*parts, bids)
    return out[:, :10]
